# baseline (speedup 1.0000x reference)
import functools

import jax
import jax.numpy as jnp
from jax import lax
from jax.experimental import pallas as pl
from jax.experimental.pallas import tpu as pltpu

_BB = 8192


def _tc_body(idx_ref, table_ref, out_ref):
    idx = idx_ref[0, 0, :]
    V = table_ref.shape[0]
    onehot = (idx[:, None] == lax.broadcasted_iota(jnp.int32, (idx.shape[0], V), 1)).astype(jnp.float32)
    out_ref[...] = jax.lax.dot_general(
        onehot, table_ref[...],
        dimension_numbers=(((1,), (0,)), ((), ())),
        preferred_element_type=jnp.float32,
    )


@functools.cache
def _make_tc(V, D, B, BB):
    NB = B // BB

    def call(idx, table):
        idx3 = idx.reshape(NB, 1, BB)
        return pl.pallas_call(
            _tc_body,
            grid=(NB,),
            in_specs=[
                pl.BlockSpec((1, 1, BB), lambda i: (i, 0, 0)),
                pl.BlockSpec((V, D), lambda i: (0, 0)),
            ],
            out_specs=pl.BlockSpec((BB, D), lambda i: (i, 0)),
            out_shape=jax.ShapeDtypeStruct((B, D), jnp.float32),
        )(idx3, table)

    return call


@jax.jit
def kernel(inputs, table):
    idx = inputs.astype(jnp.int32)
    return _make_tc(table.shape[0], table.shape[1], idx.shape[0], _BB)(idx, table)
